# baseline (speedup 1.0000x reference)
import functools

import jax
import jax.numpy as jnp
from jax import lax
from jax.experimental import pallas as pl
from jax.experimental.pallas import tpu as pltpu
from jax.experimental.pallas import tpu_sc as plsc

_EMB = 128
_D0, _D1, _D2 = 5, 6, 2
_TROWS = 64

_CHUNK = 400
_LANES = 16


def _table_body(w0_ref, w1_ref, w2_ref, t_ref):
    k = lax.broadcasted_iota(jnp.int32, (_TROWS, _EMB), 0)
    a = k // (_D1 * _D2)
    b = (k // _D2) % _D1
    c = k % _D2
    t = jnp.zeros((_TROWS, _EMB), jnp.float32)
    for r in range(_D0):
        t = t + jnp.where(a == r, w0_ref[r, :][None, :], 0.0)
    for r in range(_D1):
        t = t + jnp.where(b == r, w1_ref[r, :][None, :], 0.0)
    for r in range(_D2):
        t = t + jnp.where(c == r, w2_ref[r, :][None, :], 0.0)
    t_ref[...] = t


_combined_table = pl.pallas_call(
    _table_body,
    out_shape=jax.ShapeDtypeStruct((_TROWS, _EMB), jnp.float32),
)


@functools.cache
def _make_gather(E: int):
    info = plsc.get_sparse_core_info()
    nw = info.num_cores * info.num_subcores
    assert E % (nw * _CHUNK) == 0, E
    per_w = E // nw
    n_chunks = per_w // _CHUNK
    mesh = plsc.VectorSubcoreMesh(core_axis_name="c", subcore_axis_name="s")

    @functools.partial(
        pl.kernel,
        mesh=mesh,
        out_type=jax.ShapeDtypeStruct((E * _EMB,), jnp.float32),
        scratch_types=[
            pltpu.VMEM((_TROWS * _EMB,), jnp.float32),
            pltpu.VMEM((3 * _CHUNK,), jnp.int32),
            pltpu.VMEM((3 * _CHUNK,), jnp.int32),
            pltpu.VMEM((_CHUNK * _EMB,), jnp.float32),
            pltpu.VMEM((_CHUNK * _EMB,), jnp.float32),
            pltpu.SemaphoreType.DMA,
            pltpu.SemaphoreType.DMA,
            pltpu.SemaphoreType.DMA,
            pltpu.SemaphoreType.DMA,
        ],
        compiler_params=pltpu.CompilerParams(needs_layout_passes=False),
    )
    def gather(t_hbm, cols_hbm, out_hbm, t_v, cols0_v, cols1_v,
               rows0_v, rows1_v, sem_c0, sem_c1, sem_o0, sem_o1):
        wid = lax.axis_index("s") * info.num_cores + lax.axis_index("c")
        base_w = wid * per_w
        cbase_w = wid * n_chunks * 3 * _CHUNK
        cols = (cols0_v, cols1_v)
        rows = (rows0_v, rows1_v)
        sem_c = (sem_c0, sem_c1)
        sem_o = (sem_o0, sem_o1)
        lane = lax.iota(jnp.int32, _LANES)

        def fire_cols(j):
            return pltpu.async_copy(
                cols_hbm.at[pl.ds(cbase_w + j * 3 * _CHUNK, 3 * _CHUNK)],
                cols[j % 2], sem_c[j % 2])

        def compute_rows(j):
            cv, rv = cols[j % 2], rows[j % 2]

            def body(i, carry):
                off = i * _LANES
                a0 = cv[pl.ds(off, _LANES)]
                a1 = cv[pl.ds(_CHUNK + off, _LANES)]
                a2 = cv[pl.ds(2 * _CHUNK + off, _LANES)]
                t0 = (a0 * (_D1 * _D2) + a1 * _D2 + a2) * _EMB
                o0 = (lane + off) * _EMB

                def dbody(d, carry2):
                    val = plsc.load_gather(t_v, [t0 + d])
                    plsc.store_scatter(rv, [o0 + d], val)
                    return carry2
                lax.fori_loop(0, _EMB, dbody, 0, unroll=16)
                return carry
            lax.fori_loop(0, _CHUNK // _LANES, body, 0)

        def fire_scatter(j):
            return pltpu.async_copy(
                rows[j % 2],
                out_hbm.at[pl.ds((base_w + j * _CHUNK) * _EMB,
                                 _CHUNK * _EMB)], sem_o[j % 2])

        pltpu.sync_copy(t_hbm, t_v)
        cols_cp = {0: fire_cols(0)}
        if n_chunks > 1:
            cols_cp[1] = fire_cols(1)
        scatter_cps = {}
        for j in range(n_chunks):
            cols_cp[j].wait()
            if j >= 2:
                scatter_cps[j - 2].wait()
            compute_rows(j)
            scatter_cps[j] = fire_scatter(j)
            if j + 2 < n_chunks:
                cols_cp[j + 2] = fire_cols(j + 2)
        if n_chunks >= 2:
            scatter_cps[n_chunks - 2].wait()
        scatter_cps[n_chunks - 1].wait()

    return gather


def kernel(edge_attr, W0, W1, W2):
    E = edge_attr.shape[0]
    ea = edge_attr.astype(jnp.int32)
    packed = ea.T.reshape(3, E // _CHUNK, _CHUNK).transpose(1, 0, 2).reshape(-1)
    t = _combined_table(W0, W1, W2).reshape(-1)
    return _make_gather(E)(t, packed).reshape(E, _EMB)


# speedup vs baseline: 4.3592x; 4.3592x over previous
import functools

import jax
import jax.numpy as jnp
from jax import lax
from jax.experimental import pallas as pl
from jax.experimental.pallas import tpu as pltpu
from jax.experimental.pallas import tpu_sc as plsc

_EMB = 128
_D0, _D1, _D2 = 5, 6, 2
_TROWS = 64

_CHUNK = 400
_LANES = 16
_GPC = _CHUNK // _LANES
_CW = 3 * _CHUNK
_RW = _CHUNK * _EMB


def _table_body(w0_ref, w1_ref, w2_ref, t_ref):
    k = lax.broadcasted_iota(jnp.int32, (_TROWS, _EMB), 0)
    a = k // (_D1 * _D2)
    b = (k // _D2) % _D1
    c = k % _D2
    t = jnp.zeros((_TROWS, _EMB), jnp.float32)
    for r in range(_D0):
        t = t + jnp.where(a == r, w0_ref[r, :][None, :], 0.0)
    for r in range(_D1):
        t = t + jnp.where(b == r, w1_ref[r, :][None, :], 0.0)
    for r in range(_D2):
        t = t + jnp.where(c == r, w2_ref[r, :][None, :], 0.0)
    t_ref[...] = t


_combined_table = pl.pallas_call(
    _table_body,
    out_shape=jax.ShapeDtypeStruct((_TROWS, _EMB), jnp.float32),
)


@functools.cache
def _make_gather(E: int):
    info = plsc.get_sparse_core_info()
    nw = info.num_cores * info.num_subcores
    assert E % (nw * _CHUNK) == 0, E
    per_w = E // nw
    n_chunks = per_w // _CHUNK
    mesh = plsc.VectorSubcoreMesh(core_axis_name="c", subcore_axis_name="s")

    @functools.partial(
        pl.kernel,
        mesh=mesh,
        out_type=jax.ShapeDtypeStruct((E * _EMB,), jnp.float32),
        scratch_types=[
            pltpu.VMEM((_TROWS * _EMB,), jnp.float32),
            pltpu.VMEM((2 * _CW,), jnp.int32),
            pltpu.VMEM((2 * _RW,), jnp.float32),
            pltpu.SemaphoreType.DMA,
            pltpu.SemaphoreType.DMA,
            pltpu.SemaphoreType.DMA,
            pltpu.SemaphoreType.DMA,
        ],
        compiler_params=pltpu.CompilerParams(needs_layout_passes=False),
    )
    def gather(t_hbm, cols_hbm, out_hbm, t_v, cols_v, rows_v,
               sem_c0, sem_c1, sem_o0, sem_o1):
        wid = lax.axis_index("s") * info.num_cores + lax.axis_index("c")
        base_w = wid * per_w
        cbase_w = wid * n_chunks * _CW
        sem_c = (sem_c0, sem_c1)
        sem_o = (sem_o0, sem_o1)

        def fire_cols(j, sem):
            return pltpu.async_copy(
                cols_hbm.at[pl.ds(cbase_w + j * _CW, _CW)],
                cols_v.at[pl.ds((j % 2) * _CW if isinstance(j, int)
                                else (j & 1) * _CW, _CW)], sem)

        def drain(sem, nwords, dtype):
            pltpu.make_async_copy(
                out_hbm.at[pl.ds(0, nwords)] if dtype == jnp.float32
                else cols_hbm.at[pl.ds(0, nwords)],
                rows_v.at[pl.ds(0, nwords)] if dtype == jnp.float32
                else cols_v.at[pl.ds(0, nwords)],
                sem).wait()

        def body(gi, carry):
            j = gi // _GPC
            i = gi % _GPC
            p = j & 1
            is_first = i == 0
            is_last = i == _GPC - 1

            @pl.when(jnp.logical_and(is_first, p == 0))
            def _():
                drain(sem_c0, _CW, jnp.int32)

            @pl.when(jnp.logical_and(is_first, p == 1))
            def _():
                drain(sem_c1, _CW, jnp.int32)

            @pl.when(jnp.logical_and(is_first,
                                     jnp.logical_and(j + 1 < n_chunks,
                                                     p == 0)))
            def _():
                fire_cols(j + 1, sem_c1)

            @pl.when(jnp.logical_and(is_first,
                                     jnp.logical_and(j + 1 < n_chunks,
                                                     p == 1)))
            def _():
                fire_cols(j + 1, sem_c0)

            @pl.when(jnp.logical_and(is_first,
                                     jnp.logical_and(j >= 2, p == 0)))
            def _():
                drain(sem_o0, _RW, jnp.float32)

            @pl.when(jnp.logical_and(is_first,
                                     jnp.logical_and(j >= 2, p == 1)))
            def _():
                drain(sem_o1, _RW, jnp.float32)

            coff = p * _CW + i * _LANES
            a0 = cols_v[pl.ds(coff, _LANES)]
            a1 = cols_v[pl.ds(coff + _CHUNK, _LANES)]
            a2 = cols_v[pl.ds(coff + 2 * _CHUNK, _LANES)]
            t16 = (a0 * (_D1 * _D2) + a1 * _D2 + a2) * _EMB
            obase = p * _RW + i * (_LANES * _EMB)
            for e in range(_LANES):
                tb = t16[e]
                eb = obase + e * _EMB
                for d in range(_EMB // _LANES):
                    rows_v[pl.ds(eb + d * _LANES, _LANES)] = (
                        t_v[pl.ds(tb + d * _LANES, _LANES)])

            @pl.when(jnp.logical_and(is_last, p == 0))
            def _():
                pltpu.async_copy(
                    rows_v.at[pl.ds(0, _RW)],
                    out_hbm.at[pl.ds((base_w + j * _CHUNK) * _EMB, _RW)],
                    sem_o0)

            @pl.when(jnp.logical_and(is_last, p == 1))
            def _():
                pltpu.async_copy(
                    rows_v.at[pl.ds(_RW, _RW)],
                    out_hbm.at[pl.ds((base_w + j * _CHUNK) * _EMB, _RW)],
                    sem_o1)

            return carry

        pltpu.sync_copy(t_hbm, t_v)
        fire_cols(0, sem_c0)
        lax.fori_loop(0, n_chunks * _GPC, body, 0)
        drain(sem_o[(n_chunks - 2) % 2], _RW, jnp.float32)
        drain(sem_o[(n_chunks - 1) % 2], _RW, jnp.float32)

    return gather


def kernel(edge_attr, W0, W1, W2):
    E = edge_attr.shape[0]
    ea = edge_attr.astype(jnp.int32)
    packed = ea.T.reshape(3, E // _CHUNK, _CHUNK).transpose(1, 0, 2).reshape(-1)
    t = _combined_table(W0, W1, W2).reshape(-1)
    return _make_gather(E)(t, packed).reshape(E, _EMB)


# speedup vs baseline: 13.7460x; 3.1533x over previous
import functools

import jax
import jax.numpy as jnp
from jax import lax
from jax.experimental import pallas as pl
from jax.experimental.pallas import tpu as pltpu
from jax.experimental.pallas import tpu_sc as plsc

_EMB = 128
_D0, _D1, _D2 = 5, 6, 2
_TROWS = 64

_CHUNK = 400
_GSUB = 80
_LANES = 16


def _table_body(w0_ref, w1_ref, w2_ref, t_ref):
    k = lax.broadcasted_iota(jnp.int32, (_TROWS, _EMB), 0)
    a = k // (_D1 * _D2)
    b = (k // _D2) % _D1
    c = k % _D2
    t = jnp.zeros((_TROWS, _EMB), jnp.float32)
    for r in range(_D0):
        t = t + jnp.where(a == r, w0_ref[r, :][None, :], 0.0)
    for r in range(_D1):
        t = t + jnp.where(b == r, w1_ref[r, :][None, :], 0.0)
    for r in range(_D2):
        t = t + jnp.where(c == r, w2_ref[r, :][None, :], 0.0)
    t_ref[...] = t


_combined_table = pl.pallas_call(
    _table_body,
    out_shape=jax.ShapeDtypeStruct((_TROWS, _EMB), jnp.float32),
)


@functools.cache
def _make_gather(E: int):
    info = plsc.get_sparse_core_info()
    nw = info.num_cores * info.num_subcores
    assert E % (nw * _CHUNK) == 0, E
    per_w = E // nw
    n_chunks = per_w // _CHUNK
    mesh = plsc.VectorSubcoreMesh(core_axis_name="c", subcore_axis_name="s")

    @functools.partial(
        pl.kernel,
        mesh=mesh,
        out_type=jax.ShapeDtypeStruct((E, _EMB), jnp.float32),
        scratch_types=[
            pltpu.VMEM_SHARED((_TROWS, _EMB), jnp.float32),
            pltpu.VMEM((3 * _CHUNK,), jnp.int32),
            pltpu.VMEM((3 * _CHUNK,), jnp.int32),
            pltpu.VMEM((_CHUNK,), jnp.int32),
            pltpu.VMEM((_CHUNK,), jnp.int32),
            pltpu.VMEM((_CHUNK, _EMB), jnp.float32),
            pltpu.VMEM((_CHUNK, _EMB), jnp.float32),
            pltpu.SemaphoreType.DMA,
            pltpu.SemaphoreType.DMA,
            pltpu.SemaphoreType.DMA,
            pltpu.SemaphoreType.DMA,
            pltpu.SemaphoreType.DMA,
            pltpu.SemaphoreType.DMA,
        ],
    )
    def gather(t_hbm, cols_hbm, out_hbm, t_v, cols0_v, cols1_v,
               keys0_v, keys1_v, rows0_v, rows1_v,
               sem_c0, sem_c1, sem_g0, sem_g1, sem_o0, sem_o1):
        wid = lax.axis_index("s") * info.num_cores + lax.axis_index("c")
        base_w = wid * per_w
        cbase_w = wid * n_chunks * 3 * _CHUNK
        cols = (cols0_v, cols1_v)
        keys = (keys0_v, keys1_v)
        rows = (rows0_v, rows1_v)
        sem_c = (sem_c0, sem_c1)
        sem_g = (sem_g0, sem_g1)
        sem_o = (sem_o0, sem_o1)

        def fire_cols(j):
            return pltpu.async_copy(
                cols_hbm.at[pl.ds(cbase_w + j * 3 * _CHUNK, 3 * _CHUNK)],
                cols[j % 2], sem_c[j % 2])

        def compute_keys(j):
            cv, kv = cols[j % 2], keys[j % 2]

            def body(i, carry):
                off = i * _LANES
                a0 = cv[pl.ds(off, _LANES)]
                a1 = cv[pl.ds(_CHUNK + off, _LANES)]
                a2 = cv[pl.ds(2 * _CHUNK + off, _LANES)]
                kv[pl.ds(off, _LANES)] = a0 * (_D1 * _D2) + a1 * _D2 + a2
                return carry
            lax.fori_loop(0, _CHUNK // _LANES, body, 0)

        def fire_gathers(j):
            return [
                pltpu.async_copy(
                    t_v.at[keys[j % 2].at[pl.ds(g * _GSUB, _GSUB)]],
                    rows[j % 2].at[pl.ds(g * _GSUB, _GSUB)], sem_g[j % 2])
                for g in range(_CHUNK // _GSUB)
            ]

        def fire_scatter(j):
            return pltpu.async_copy(
                rows[j % 2],
                out_hbm.at[pl.ds(base_w + j * _CHUNK, _CHUNK)], sem_o[j % 2])

        @pl.when(lax.axis_index("s") == 0)
        def _():
            pltpu.sync_copy(t_hbm, t_v)
        plsc.subcore_barrier()
        cols_cp = {0: fire_cols(0)}
        if n_chunks > 1:
            cols_cp[1] = fire_cols(1)
        cols_cp[0].wait()
        compute_keys(0)
        gather_cps = {0: fire_gathers(0)}
        scatter_cps = {}
        for j in range(n_chunks):
            if j + 2 < n_chunks:
                cols_cp[j + 2] = fire_cols(j + 2)
            if j + 1 < n_chunks:
                cols_cp[j + 1].wait()
                compute_keys(j + 1)
                if j >= 1:
                    scatter_cps[j - 1].wait()
                gather_cps[j + 1] = fire_gathers(j + 1)
            for cp in gather_cps[j]:
                cp.wait()
            scatter_cps[j] = fire_scatter(j)
        if n_chunks >= 2:
            scatter_cps[n_chunks - 2].wait()
        scatter_cps[n_chunks - 1].wait()

    return gather


def kernel(edge_attr, W0, W1, W2):
    E = edge_attr.shape[0]
    ea = edge_attr.astype(jnp.int32)
    packed = ea.T.reshape(3, E // _CHUNK, _CHUNK).transpose(1, 0, 2).reshape(-1)
    t = _combined_table(W0, W1, W2)
    return _make_gather(E)(t, packed)
